# baseline (speedup 1.0000x reference)
import functools

import numpy as np
import jax
import jax.numpy as jnp
from jax import lax
from jax.experimental import pallas as pl
from jax.experimental.pallas import tpu as pltpu
from jax.experimental.pallas import tpu_sc as plsc

_SIZE_VERTEX = 100000
_D = 64


def _build_leaves(size_vertex):
    leaf = []

    def rec(tl, tr, v):
        if tl == tr:
            leaf.append(v)
            return
        tm = (tl + tr) >> 1
        rec(tl, tm, 2 * v)
        rec(tm + 1, tr, 2 * v + 1)

    rec(1, size_vertex + 1, 1)
    return leaf


_LEAF = np.asarray(_build_leaves(_SIZE_VERTEX), dtype=np.int32)
_MAX_BITS = int(_LEAF.max()).bit_length()
_NLEV = _MAX_BITS - 1
_NPAD = 32
_NSUB = 16

_PERM_TBL = np.zeros((4, 128), dtype=np.int32)
for _k in range(4):
    _PERM_TBL[_k, :16] = np.arange(16, dtype=np.int32) ^ (1 << _k)

_mesh = plsc.VectorSubcoreMesh(
    core_axis_name="c", subcore_axis_name="s", num_cores=1)


@functools.partial(
    pl.kernel,
    out_type=jax.ShapeDtypeStruct((16,), jnp.float32),
    mesh=_mesh,
    compiler_params=pltpu.CompilerParams(use_tc_tiling_on_sc=True),
    scratch_types=[
        pltpu.VMEM((8, 128), jnp.int32),
        pltpu.VMEM((8, 128), jnp.float32),
        pltpu.VMEM((_D, 128), jnp.float32),
        pltpu.VMEM((_D, 128), jnp.float32),
        pltpu.VMEM((_D, 128), jnp.float32),
        pltpu.VMEM((16,), jnp.float32),
        pltpu.VMEM((_NPAD, 16), jnp.float32),
        pltpu.VMEM_SHARED((_NPAD, 16), jnp.float32),
        pltpu.SemaphoreType.DMA,
    ],
)
def _hs_path_kernel(wt_hbm, enct_hbm, idx3_hbm, par3_hbm, out_hbm,
                    idxt_v, part_v, w0_v, w1_v, e_v, fac_v, allfac_v,
                    shared, sem):
    cid = lax.axis_index("c")
    sid = lax.axis_index("s")

    @pl.when(cid == 0)
    def _():
        pltpu.sync_copy(idx3_hbm.at[sid], idxt_v)
        pltpu.sync_copy(par3_hbm.at[sid], part_v)
        ivec = idxt_v[0, pl.ds(0, 16)]
        wb0 = pl.multiple_of(ivec[0], 128)
        wb1 = pl.multiple_of(ivec[1], 128)
        eb = pl.multiple_of(ivec[2], 128)
        choh = pl.multiple_of(ivec[3], 16)
        ch0 = pl.multiple_of(ivec[4], 16)
        ch1 = pl.multiple_of(ivec[5], 16)
        lane0 = jnp.broadcast_to(ivec[6], (16,))
        lane1 = jnp.broadcast_to(ivec[7], (16,))
        perms = [idxt_v[1 + k, pl.ds(0, 16)] for k in range(4)]

        cp0 = pltpu.async_copy(wt_hbm.at[:, pl.ds(wb0, 128)], w0_v, sem)
        cp1 = pltpu.async_copy(wt_hbm.at[:, pl.ds(wb1, 128)], w1_v, sem)
        cpe = pltpu.async_copy(enct_hbm.at[:, pl.ds(eb, 128)], e_v, sem)
        cp0.wait()
        cp1.wait()
        cpe.wait()

        d0 = None
        d1 = None
        for c in range(_D):
            hc = e_v[c, pl.ds(choh, 16)]
            w0c = w0_v[c, pl.ds(ch0, 16)]
            w1c = w1_v[c, pl.ds(ch1, 16)]
            w0b = w0c.at[lane0].get(mode="promise_in_bounds")
            w1b = w1c.at[lane1].get(mode="promise_in_bounds")
            t0 = hc * w0b
            t1 = hc * w1b
            d0 = t0 if d0 is None else d0 + t0
            d1 = t1 if d1 is None else d1 + t1

        ohh = part_v[0, pl.ds(0, 16)]
        d0 = d0 * ohh
        d1 = d1 * ohh
        for k in range(4):
            d0 = d0 + d0.at[perms[k]].get(mode="promise_in_bounds")
            d1 = d1 + d1.at[perms[k]].get(mode="promise_in_bounds")

        pvec = part_v[1, pl.ds(0, 16)]
        m0, va0, m1, va1 = pvec[0], pvec[1], pvec[2], pvec[3]
        f0 = va0 * (1.0 / (1.0 + jnp.exp(-(m0 * d0)))) + (1.0 - va0)
        f1 = va1 * (1.0 / (1.0 + jnp.exp(-(m1 * d1)))) + (1.0 - va1)

        fac_v[pl.ds(0, 16)] = f0
        pltpu.sync_copy(fac_v, shared.at[sid])
        fac_v[pl.ds(0, 16)] = f1
        pltpu.sync_copy(fac_v, shared.at[sid + _NSUB])
        plsc.subcore_barrier()

        @pl.when(sid == 0)
        def _():
            pltpu.sync_copy(shared, allfac_v)
            p = allfac_v[0, pl.ds(0, 16)][0]
            for r in range(1, _NPAD):
                p = p * allfac_v[r, pl.ds(0, 16)][0]
            fac_v[pl.ds(0, 16)] = jnp.broadcast_to(p, (16,))
            pltpu.sync_copy(fac_v, out_hbm)


def kernel(encoder, W, v_i, v_j):
    new_node = jnp.asarray(_LEAF).at[jnp.asarray(v_i, jnp.int32)].get(
        mode="promise_in_bounds")
    vj = jnp.asarray(v_j, jnp.int32)

    shp = (_NSUB, 8, 128)
    sub = lax.broadcasted_iota(jnp.int32, shp, 0)
    row = lax.broadcasted_iota(jnp.int32, shp, 1)
    lane = lax.broadcasted_iota(jnp.int32, shp, 2)

    lev_a = sub
    lev_b = sub + _NSUB
    sh_a = (_MAX_BITS - 1) - lev_a
    par_a = lax.shift_right_logical(new_node, sh_a)
    chd_a = lax.shift_right_logical(new_node, sh_a - 1)
    is_b = lev_b < _NLEV
    par_b = jnp.where(is_b, lax.shift_right_logical(new_node, 1), 0)
    chd_b = jnp.where(is_b, new_node, 0)

    eb = (vj >> 7) << 7
    choh = ((vj & 127) >> 4) << 4
    lhm = vj & 15

    perm = lane ^ (1 << jnp.maximum(row - 1, 0))
    row0 = jnp.where(lane == 0, (par_a >> 7) << 7,
           jnp.where(lane == 1, (par_b >> 7) << 7,
           jnp.where(lane == 2, eb,
           jnp.where(lane == 3, choh,
           jnp.where(lane == 4, ((par_a & 127) >> 4) << 4,
           jnp.where(lane == 5, ((par_b & 127) >> 4) << 4,
           jnp.where(lane == 6, par_a & 15,
           jnp.where(lane == 7, par_b & 15, 0))))))))
    idx3 = jnp.where(
        row == 0, row0,
        jnp.where((row >= 1) & (row <= 4) & (lane < 16), perm, 0))

    mult_a = jnp.where(chd_a % 2 == 0, 1.0, -1.0)
    mult_b = jnp.where(chd_b % 2 == 0, 1.0, -1.0)
    valid_a = (par_a >= 2).astype(jnp.float32)
    valid_b = ((par_b >= 2) & is_b).astype(jnp.float32)
    par_row1 = jnp.where(lane == 0, mult_a,
               jnp.where(lane == 1, valid_a,
               jnp.where(lane == 2, mult_b,
               jnp.where(lane == 3, valid_b, 0.0))))
    par3 = jnp.where(
        (row == 0) & (lane == lhm), 1.0,
        jnp.where(row == 1, par_row1, 0.0))

    out = _hs_path_kernel(W.T, encoder.T, idx3, par3)
    return out[0:1]


# speedup vs baseline: 1.0392x; 1.0392x over previous
import functools

import numpy as np
import jax
import jax.numpy as jnp
from jax import lax
from jax.experimental import pallas as pl
from jax.experimental.pallas import tpu as pltpu
from jax.experimental.pallas import tpu_sc as plsc

_SIZE_VERTEX = 100000
_D = 64


def _build_leaves(size_vertex):
    leaf = []

    def rec(tl, tr, v):
        if tl == tr:
            leaf.append(v)
            return
        tm = (tl + tr) >> 1
        rec(tl, tm, 2 * v)
        rec(tm + 1, tr, 2 * v + 1)

    rec(1, size_vertex + 1, 1)
    return leaf


_LEAF = np.asarray(_build_leaves(_SIZE_VERTEX), dtype=np.int32)
_MAX_BITS = int(_LEAF.max()).bit_length()
_NLEV = _MAX_BITS - 1
_NPAD = 32
_NSUB = 16

_mesh = plsc.VectorSubcoreMesh(
    core_axis_name="c", subcore_axis_name="s", num_cores=1)


@functools.partial(
    pl.kernel,
    out_type=jax.ShapeDtypeStruct((16,), jnp.float32),
    mesh=_mesh,
    compiler_params=pltpu.CompilerParams(use_tc_tiling_on_sc=True),
    scratch_types=[
        pltpu.VMEM((8, 128), jnp.int32),
        pltpu.VMEM((_D, 128), jnp.float32),
        pltpu.VMEM((_D, 128), jnp.float32),
        pltpu.VMEM((_D, 128), jnp.float32),
        pltpu.VMEM((16,), jnp.float32),
        pltpu.VMEM((_NPAD, 16), jnp.float32),
        pltpu.VMEM_SHARED((_NPAD, 16), jnp.float32),
        pltpu.SemaphoreType.DMA,
    ],
)
def _hs_path_kernel(wt_hbm, enct_hbm, idx3_hbm, out_hbm,
                    idxt_v, w0_v, w1_v, e_v, fac_v, allfac_v,
                    shared, sem):
    cid = lax.axis_index("c")
    sid = lax.axis_index("s")

    @pl.when(cid == 0)
    def _():
        pltpu.sync_copy(idx3_hbm.at[sid], idxt_v)
        ivec = idxt_v[0, pl.ds(0, 16)]
        wb0 = pl.multiple_of(ivec[0], 128)
        wb1 = pl.multiple_of(ivec[1], 128)
        eb = pl.multiple_of(ivec[2], 128)
        choh = pl.multiple_of(ivec[3], 16)
        ch0 = pl.multiple_of(ivec[4], 16)
        ch1 = pl.multiple_of(ivec[5], 16)
        lane0 = jnp.broadcast_to(ivec[6], (16,))
        lane1 = jnp.broadcast_to(ivec[7], (16,))
        laneh = jnp.broadcast_to(ivec[8], (16,))

        cp0 = pltpu.async_copy(wt_hbm.at[:, pl.ds(wb0, 128)], w0_v, sem)
        cp1 = pltpu.async_copy(wt_hbm.at[:, pl.ds(wb1, 128)], w1_v, sem)
        cpe = pltpu.async_copy(enct_hbm.at[:, pl.ds(eb, 128)], e_v, sem)
        cp0.wait()
        cp1.wait()
        cpe.wait()

        d0 = None
        d1 = None
        for c in range(_D):
            hc = e_v[c, pl.ds(choh, 16)]
            w0c = w0_v[c, pl.ds(ch0, 16)]
            w1c = w1_v[c, pl.ds(ch1, 16)]
            w0b = w0c.at[lane0].get(mode="promise_in_bounds")
            w1b = w1c.at[lane1].get(mode="promise_in_bounds")
            t0 = hc * w0b
            t1 = hc * w1b
            d0 = t0 if d0 is None else d0 + t0
            d1 = t1 if d1 is None else d1 + t1

        dd0 = d0.at[laneh].get(mode="promise_in_bounds")
        dd1 = d1.at[laneh].get(mode="promise_in_bounds")
        m0 = jnp.broadcast_to(ivec[9], (16,)).astype(jnp.float32)
        va0 = jnp.broadcast_to(ivec[10], (16,)).astype(jnp.float32)
        m1 = jnp.broadcast_to(ivec[11], (16,)).astype(jnp.float32)
        va1 = jnp.broadcast_to(ivec[12], (16,)).astype(jnp.float32)
        f0 = va0 * (1.0 / (1.0 + jnp.exp(-(m0 * dd0)))) + (1.0 - va0)
        f1 = va1 * (1.0 / (1.0 + jnp.exp(-(m1 * dd1)))) + (1.0 - va1)

        fac_v[pl.ds(0, 16)] = f0
        pltpu.sync_copy(fac_v, shared.at[sid])
        fac_v[pl.ds(0, 16)] = f1
        pltpu.sync_copy(fac_v, shared.at[sid + _NSUB])
        plsc.subcore_barrier()

        @pl.when(sid == 0)
        def _():
            pltpu.sync_copy(shared, allfac_v)
            p = allfac_v[0, pl.ds(0, 16)][0]
            for r in range(1, _NPAD):
                p = p * allfac_v[r, pl.ds(0, 16)][0]
            fac_v[pl.ds(0, 16)] = jnp.broadcast_to(p, (16,))
            pltpu.sync_copy(fac_v, out_hbm)


def kernel(encoder, W, v_i, v_j):
    new_node = jnp.asarray(_LEAF).at[jnp.asarray(v_i, jnp.int32)].get(
        mode="promise_in_bounds")
    vj = jnp.asarray(v_j, jnp.int32)

    shp = (_NSUB, 8, 128)
    sub = lax.broadcasted_iota(jnp.int32, shp, 0)
    row = lax.broadcasted_iota(jnp.int32, shp, 1)
    lane = lax.broadcasted_iota(jnp.int32, shp, 2)

    lev_a = sub
    lev_b = sub + _NSUB
    sh_a = (_MAX_BITS - 1) - lev_a
    par_a = lax.shift_right_logical(new_node, sh_a)
    chd_a = lax.shift_right_logical(new_node, sh_a - 1)
    is_b = lev_b < _NLEV
    par_b = jnp.where(is_b, lax.shift_right_logical(new_node, 1), 0)
    chd_b = jnp.where(is_b, new_node, 0)

    eb = (vj >> 7) << 7
    choh = ((vj & 127) >> 4) << 4
    lhm = vj & 15

    mult_a = jnp.where(chd_a % 2 == 0, 1, -1)
    mult_b = jnp.where(chd_b % 2 == 0, 1, -1)
    valid_a = (par_a >= 2).astype(jnp.int32)
    valid_b = ((par_b >= 2) & is_b).astype(jnp.int32)
    row0 = jnp.where(lane == 0, (par_a >> 7) << 7,
           jnp.where(lane == 1, (par_b >> 7) << 7,
           jnp.where(lane == 2, eb,
           jnp.where(lane == 3, choh,
           jnp.where(lane == 4, ((par_a & 127) >> 4) << 4,
           jnp.where(lane == 5, ((par_b & 127) >> 4) << 4,
           jnp.where(lane == 6, par_a & 15,
           jnp.where(lane == 7, par_b & 15,
           jnp.where(lane == 8, lhm,
           jnp.where(lane == 9, mult_a,
           jnp.where(lane == 10, valid_a,
           jnp.where(lane == 11, mult_b,
           jnp.where(lane == 12, valid_b, 0)))))))))))))
    idx3 = jnp.where(row == 0, row0, 0)

    out = _hs_path_kernel(W.T, encoder.T, idx3)
    return out[0:1]


# speedup vs baseline: 1.1861x; 1.1414x over previous
import functools

import numpy as np
import jax
import jax.numpy as jnp
from jax import lax
from jax.experimental import pallas as pl
from jax.experimental.pallas import tpu as pltpu
from jax.experimental.pallas import tpu_sc as plsc

_SIZE_VERTEX = 100000
_D = 64


def _build_leaves(size_vertex):
    leaf = []

    def rec(tl, tr, v):
        if tl == tr:
            leaf.append(v)
            return
        tm = (tl + tr) >> 1
        rec(tl, tm, 2 * v)
        rec(tm + 1, tr, 2 * v + 1)

    rec(1, size_vertex + 1, 1)
    return leaf


_LEAF = np.asarray(_build_leaves(_SIZE_VERTEX), dtype=np.int32)
_MAX_BITS = int(_LEAF.max()).bit_length()
_NLEV = _MAX_BITS - 1
_NPAD = 32
_NSUB = 16
_LEAF_PAD = np.zeros((-(-_LEAF.size // 128)) * 128, dtype=np.int32)
_LEAF_PAD[:_LEAF.size] = _LEAF
_LEAF_PAD = _LEAF_PAD.reshape(-1, 128)

_mesh = plsc.VectorSubcoreMesh(
    core_axis_name="c", subcore_axis_name="s", num_cores=1)


def _bcast(x):
    return jnp.broadcast_to(x, (16,))


@functools.partial(
    pl.kernel,
    out_type=jax.ShapeDtypeStruct((16,), jnp.float32),
    mesh=_mesh,
    compiler_params=pltpu.CompilerParams(use_tc_tiling_on_sc=True),
    scratch_types=[
        pltpu.VMEM((8, 128), jnp.int32),
        pltpu.VMEM((1, 128), jnp.int32),
        pltpu.VMEM((_D, 128), jnp.float32),
        pltpu.VMEM((_D, 128), jnp.float32),
        pltpu.VMEM((_D, 128), jnp.float32),
        pltpu.VMEM((16,), jnp.float32),
        pltpu.VMEM((_NPAD, 16), jnp.float32),
        pltpu.VMEM_SHARED((_NPAD, 16), jnp.float32),
        pltpu.SemaphoreType.DMA,
    ],
)
def _hs_path_kernel(wt_hbm, enct_hbm, leaf_hbm, scal_hbm, out_hbm,
                    scal_v, leaf_v, w0_v, w1_v, e_v, fac_v, allfac_v,
                    shared, sem):
    sid = lax.axis_index("s")

    pltpu.sync_copy(scal_hbm, scal_v)
    svec = scal_v[0, pl.ds(0, 16)]
    vi = svec[0]
    vj = svec[1]

    pltpu.async_copy(leaf_hbm.at[pl.ds(vi >> 7, 1)], leaf_v, sem).wait()
    lch = pl.multiple_of(((vi & 127) >> 4) * 16, 16)
    lvec = leaf_v[0, pl.ds(lch, 16)]
    lm = vi & 15
    nn = lvec[15]
    for i in range(15):
        nn = jnp.where(lm == i, lvec[i], nn)

    sh = (_MAX_BITS - 1) - sid
    pa = lax.shift_right_logical(nn, sh)
    ca = lax.shift_right_logical(nn, sh - 1)
    is_b = sid == 0
    pb = jnp.where(is_b, lax.shift_right_logical(nn, 1), 0)
    cb = jnp.where(is_b, nn, 0)

    wb0 = pl.multiple_of((pa >> 7) * 128, 128)
    wb1 = pl.multiple_of((pb >> 7) * 128, 128)
    eb = pl.multiple_of((vj >> 7) * 128, 128)
    cp0 = pltpu.async_copy(wt_hbm.at[:, pl.ds(wb0, 128)], w0_v, sem)
    cp1 = pltpu.async_copy(wt_hbm.at[:, pl.ds(wb1, 128)], w1_v, sem)
    cpe = pltpu.async_copy(enct_hbm.at[:, pl.ds(eb, 128)], e_v, sem)

    choh = pl.multiple_of(((vj & 127) >> 4) * 16, 16)
    ch0 = pl.multiple_of(((pa & 127) >> 4) * 16, 16)
    ch1 = pl.multiple_of(((pb & 127) >> 4) * 16, 16)
    lane0 = _bcast(pa & 15)
    lane1 = _bcast(pb & 15)
    laneh = _bcast(vj & 15)
    cp0.wait()
    cp1.wait()
    cpe.wait()

    d0 = None
    d1 = None
    for c in range(_D):
        hc = e_v[c, pl.ds(choh, 16)]
        w0c = w0_v[c, pl.ds(ch0, 16)]
        w1c = w1_v[c, pl.ds(ch1, 16)]
        w0b = w0c.at[lane0].get(mode="promise_in_bounds")
        w1b = w1c.at[lane1].get(mode="promise_in_bounds")
        t0 = hc * w0b
        t1 = hc * w1b
        d0 = t0 if d0 is None else d0 + t0
        d1 = t1 if d1 is None else d1 + t1

    dd0 = d0.at[laneh].get(mode="promise_in_bounds")
    dd1 = d1.at[laneh].get(mode="promise_in_bounds")
    m0 = _bcast(jnp.where(ca % 2 == 0, 1, -1)).astype(jnp.float32)
    m1 = _bcast(jnp.where(cb % 2 == 0, 1, -1)).astype(jnp.float32)
    va0 = _bcast((pa >= 2).astype(jnp.int32)).astype(jnp.float32)
    va1 = _bcast(((pb >= 2) & is_b).astype(jnp.int32)).astype(jnp.float32)
    f0 = va0 * (1.0 / (1.0 + jnp.exp(-(m0 * dd0)))) + (1.0 - va0)
    f1 = va1 * (1.0 / (1.0 + jnp.exp(-(m1 * dd1)))) + (1.0 - va1)

    fac_v[pl.ds(0, 16)] = f0
    pltpu.sync_copy(fac_v, shared.at[sid])
    fac_v[pl.ds(0, 16)] = f1
    pltpu.sync_copy(fac_v, shared.at[sid + _NSUB])
    plsc.subcore_barrier()

    @pl.when(sid == 0)
    def _():
        pltpu.sync_copy(shared, allfac_v)
        prod = allfac_v[0, pl.ds(0, 16)]
        for r in range(1, _NPAD):
            prod = prod * allfac_v[r, pl.ds(0, 16)]
        fac_v[pl.ds(0, 16)] = prod
        pltpu.sync_copy(fac_v, out_hbm)


def kernel(encoder, W, v_i, v_j):
    vi = jnp.asarray(v_i, jnp.int32)
    vj = jnp.asarray(v_j, jnp.int32)
    row = lax.broadcasted_iota(jnp.int32, (8, 128), 0)
    lane = lax.broadcasted_iota(jnp.int32, (8, 128), 1)
    scal = jnp.where((row == 0) & (lane == 0), vi,
                     jnp.where((row == 0) & (lane == 1), vj, 0))
    out = _hs_path_kernel(W.T, encoder.T, jnp.asarray(_LEAF_PAD), scal)
    return out[0:1]
